# baseline (speedup 1.0000x reference)
import functools

import jax
import jax.numpy as jnp
import numpy as np
from jax import lax
from jax.experimental import pallas as pl
from jax.experimental.pallas import tpu as pltpu
from jax.experimental.pallas import tpu_sc as plsc

_TEMPERATURE = 0.8
_ROT_A = (13, 15, 26, 6)
_ROT_B = (17, 29, 16, 24)
_KEY_HI = np.uint32(0)
_KEY_LO = np.uint32(42)
_TINY = np.float32(1.1754943508222875e-38)
_NEG = np.float32(-np.inf)

_NC = 2
_NS = 16
_NSUB = _NC * _NS
_C = 384
_GRP = 160
_KCAP = 50


def _rotl(x, r):
    return (x << np.uint32(r)) | (x >> np.uint32(32 - r))


def _threefry_bits(flat_u32):
    ks0 = _KEY_HI
    ks1 = _KEY_LO
    ks2 = np.uint32(ks0 ^ ks1 ^ np.uint32(0x1BD11BDA))
    x0 = jnp.zeros_like(flat_u32) + ks0
    x1 = flat_u32 + ks1

    def four(x0, x1, rots):
        for r in rots:
            x0 = x0 + x1
            x1 = _rotl(x1, r)
            x1 = x1 ^ x0
        return x0, x1

    x0, x1 = four(x0, x1, _ROT_A)
    x0 = x0 + ks1
    x1 = x1 + np.uint32(ks2 + np.uint32(1))
    x0, x1 = four(x0, x1, _ROT_B)
    x0 = x0 + ks2
    x1 = x1 + np.uint32(ks0 + np.uint32(2))
    x0, x1 = four(x0, x1, _ROT_A)
    x0 = x0 + ks0
    x1 = x1 + np.uint32(ks1 + np.uint32(3))
    x0, x1 = four(x0, x1, _ROT_B)
    x0 = x0 + ks1
    x1 = x1 + np.uint32(ks2 + np.uint32(4))
    x0, x1 = four(x0, x1, _ROT_A)
    x0 = x0 + ks2
    x1 = x1 + np.uint32(ks0 + np.uint32(5))
    return x0 ^ x1


def _gumbel_from_flat(flat_i32):
    bits = _threefry_bits(flat_i32.astype(jnp.uint32))
    float_bits = (bits >> np.uint32(9)) | np.uint32(0x3F800000)
    floats = jax.lax.bitcast_convert_type(float_bits, jnp.float32) - np.float32(1.0)
    u = jnp.maximum(_TINY, floats * np.float32(np.float32(1.0) - _TINY) + _TINY)
    return -jnp.log(-jnp.log(u))


def _extract_kth(cval, kidx, neg_vec):
    def body(_, carry):
        cnt, tcur, tfin = carry
        mv = [neg_vec] * 4
        for j in range(_C // 16):
            v = cval[pl.ds(j * 16, 16)]
            mv[j % 4] = jnp.maximum(mv[j % 4], jnp.where(v < tcur, v, neg_vec))
        m = jnp.max(jnp.maximum(jnp.maximum(mv[0], mv[1]), jnp.maximum(mv[2], mv[3])))
        cv = [jnp.zeros((16,), jnp.float32)] * 4
        for j in range(_C // 16):
            v = cval[pl.ds(j * 16, 16)]
            cv[j % 4] = cv[j % 4] + (v == m).astype(jnp.float32)
        cc = jnp.sum((cv[0] + cv[1]) + (cv[2] + cv[3]))
        done = cnt > kidx
        newcnt = cnt + cc
        hit = jnp.logical_and(jnp.logical_not(done), newcnt > kidx)
        tfin = jnp.where(hit, m, tfin)
        cnt = jnp.where(done, cnt, newcnt)
        tcur = jnp.where(done, tcur, m)
        return cnt, tcur, tfin

    _, _, t = lax.fori_loop(
        0, _KCAP, body, (np.float32(0.0), np.float32(np.inf), _NEG)
    )
    return t


def _tighten(cval, ccol, kidx, pos_s, tau_s, lane, neg_vec):
    kth = _extract_kth(cval, kidx, neg_vec)
    pos2 = 0
    for j in range(_C // 16):
        v = cval[pl.ds(j * 16, 16)]
        cols = ccol[pl.ds(j * 16, 16)]
        mk = v >= kth
        cj = jnp.sum(mk.astype(jnp.float32)).astype(jnp.int32)

        @pl.when(cj > 0)
        def _(v=v, cols=cols, mk=mk, pos2=pos2):
            plsc.store_compressed(cval.at[pl.ds(pos2, 16)], v, mask=mk)
            plsc.store_compressed(ccol.at[pl.ds(pos2, 16)], cols, mask=mk)

        pos2 = pos2 + cj
    for j in range(_C // 16):
        lidx = lane + np.int32(j * 16)
        v = cval[pl.ds(j * 16, 16)]
        cval[pl.ds(j * 16, 16)] = jnp.where(lidx < pos2, v, neg_vec)
    pos_s[0] = pos2
    tau_s[0] = kth


def _build_sc_select(batch, vocab):
    rows_per = batch // _NSUB
    n_groups = vocab // _GRP
    mesh = plsc.VectorSubcoreMesh(
        core_axis_name="c", subcore_axis_name="s", num_cores=_NC, num_subcores=_NS
    )

    @functools.partial(
        pl.kernel,
        mesh=mesh,
        out_type=[
            jax.ShapeDtypeStruct((batch * _C,), jnp.float32),
            jax.ShapeDtypeStruct((batch * _C,), jnp.int32),
            jax.ShapeDtypeStruct((batch * 16,), jnp.float32),
        ],

        scratch_types=[
            pltpu.VMEM((vocab,), jnp.float32),
            pltpu.VMEM((_C,), jnp.float32),
            pltpu.VMEM((_C,), jnp.int32),
            pltpu.VMEM((16,), jnp.float32),
            pltpu.VMEM((16,), jnp.float32),
            pltpu.SMEM((8,), jnp.int32),
            pltpu.SMEM((8,), jnp.float32),
        ],
        compiler_params=pltpu.CompilerParams(needs_layout_passes=False),
    )
    def sel(
        logits_hbm,
        kvec_hbm,
        val_hbm,
        col_hbm,
        stats_hbm,
        rowbuf,
        cval,
        ccol,
        kvec_v,
        stage_f,
        pos_s,
        tau_s,
    ):
        wid = lax.axis_index("s") * _NC + lax.axis_index("c")
        pltpu.sync_copy(kvec_hbm, kvec_v)
        kidx = jnp.max(kvec_v[...])
        lane = lax.iota(jnp.int32, 16)
        neg_vec = jnp.full((16,), _NEG, jnp.float32)
        zero_vec = jnp.zeros((16,), jnp.int32)

        def row_body(rr, _carry):
            r = wid * rows_per + rr
            pltpu.sync_copy(logits_hbm.at[pl.ds(r * vocab, vocab)], rowbuf)
            for j in range(_C // 16):
                cval[pl.ds(j * 16, 16)] = neg_vec
                ccol[pl.ds(j * 16, 16)] = zero_vec
            for j in range(_GRP // 16):
                off = j * 16
                cval[pl.ds(off, 16)] = rowbuf[pl.ds(off, 16)]
                ccol[pl.ds(off, 16)] = lane + np.int32(off)
            pos_s[0] = np.int32(_GRP)
            tau_s[0] = _NEG
            _tighten(cval, ccol, kidx, pos_s, tau_s, lane, neg_vec)

            def group(gi, _):
                tau = tau_s[0]
                gmaxes = []
                for half in range(8):
                    hbase = (1 + 8 * gi + half) * _GRP
                    mv = [neg_vec] * 4
                    for j in range(_GRP // 16):
                        v = rowbuf[pl.ds(hbase + j * 16, 16)]
                        mv[j % 4] = jnp.maximum(mv[j % 4], v)
                    gmaxes.append(
                        jnp.max(
                            jnp.maximum(
                                jnp.maximum(mv[0], mv[1]), jnp.maximum(mv[2], mv[3])
                            )
                        )
                    )

                for half in range(8):

                    @pl.when(gmaxes[half] >= tau)
                    def _(half=half):
                        base = (1 + 8 * gi + half) * _GRP
                        vs = []
                        mks = []
                        cs = []
                        for j in range(_GRP // 16):
                            v = rowbuf[pl.ds(base + j * 16, 16)]
                            mk = v >= tau
                            vs.append(v)
                            mks.append(mk)
                            cs.append(
                                jnp.sum(mk.astype(jnp.float32)).astype(jnp.int32)
                            )
                        ctot = cs[0]
                        for j in range(1, _GRP // 16):
                            ctot = ctot + cs[j]
                        p0 = pos_s[0]

                        @pl.when(p0 + ctot > _C - 16)
                        def _():
                            _tighten(cval, ccol, kidx, pos_s, tau_s, lane, neg_vec)

                        p = pos_s[0]

                        @pl.when(p + ctot <= _C - 16)
                        def _():
                            off = p
                            for j in range(_GRP // 16):
                                cols = lane + (base + np.int32(j * 16))
                                plsc.store_compressed(
                                    cval.at[pl.ds(off, 16)], vs[j], mask=mks[j]
                                )
                                plsc.store_compressed(
                                    ccol.at[pl.ds(off, 16)], cols, mask=mks[j]
                                )
                                off = off + cs[j]
                            pos_s[0] = off

                return 0

            lax.fori_loop(0, (n_groups - 1) // 8, group, 0)

            mv = [neg_vec] * 4
            for j in range(_C // 16):
                mv[j % 4] = jnp.maximum(mv[j % 4], cval[pl.ds(j * 16, 16)])
            m_row = jnp.max(
                jnp.maximum(jnp.maximum(mv[0], mv[1]), jnp.maximum(mv[2], mv[3]))
            )
            t_row = _extract_kth(cval, kidx, neg_vec)
            stage_f[...] = jnp.where(
                lane == 0, t_row, jnp.where(lane == 1, m_row, np.float32(0.0))
            )
            pltpu.sync_copy(cval, val_hbm.at[pl.ds(r * _C, _C)])
            pltpu.sync_copy(ccol, col_hbm.at[pl.ds(r * _C, _C)])
            pltpu.sync_copy(stage_f, stats_hbm.at[pl.ds(r * 16, 16)])
            return 0

        lax.fori_loop(0, rows_per, row_body, 0)

    return sel


def _tc_finish_body(
    vocab, logits_ref, cval_ref, ccol_ref, tcol_ref, mcol_ref, probs_ref, idx_ref
):
    temp = np.float32(_TEMPERATURE)
    rows = logits_ref.shape[0]
    t_s = tcol_ref[:, :1] / temp
    m_s = mcol_ref[:, :1] / temp

    sv = cval_ref[...] / temp
    mask = sv >= t_s
    e = jnp.where(mask, jnp.exp(sv - m_s), np.float32(0.0))
    s_row = jnp.sum(e, axis=1, keepdims=True)

    col = ccol_ref[...]
    pid = pl.program_id(0)
    row = jax.lax.broadcasted_iota(jnp.int32, col.shape, 0) + pid * rows
    flat = row * vocab + col
    g = _gumbel_from_flat(flat)
    score = jnp.where(mask, sv + g, _NEG)
    sm = jnp.max(score, axis=1, keepdims=True)
    first = jnp.min(
        jnp.where(score == sm, col, np.int32(2**30)), axis=1, keepdims=True
    )
    idx_ref[...] = jnp.broadcast_to(first, idx_ref.shape).astype(jnp.int32)

    scaled = logits_ref[...] / temp
    probs_ref[...] = jnp.where(
        scaled >= t_s, jnp.exp(scaled - m_s) / s_row, np.float32(0.0)
    )


@jax.jit
def _run(logits, kidx):
    batch, vocab = logits.shape
    rows = 8
    sel = _build_sc_select(batch, vocab)
    kvec = jnp.full((16,), kidx, jnp.int32).astype(jnp.float32)
    val, colb, stats = sel(logits.reshape(-1), kvec)
    stats2 = stats.reshape(batch, 16)
    tcol = jnp.broadcast_to(stats2[:, 0:1], (batch, 128))
    mcol = jnp.broadcast_to(stats2[:, 1:2], (batch, 128))
    cval2 = val.reshape(batch, _C)
    ccol2 = colb.reshape(batch, _C)

    probs, idx = pl.pallas_call(
        functools.partial(_tc_finish_body, vocab),
        grid=(batch // rows,),
        in_specs=[
            pl.BlockSpec((rows, vocab), lambda i: (i, 0)),
            pl.BlockSpec((rows, _C), lambda i: (i, 0)),
            pl.BlockSpec((rows, _C), lambda i: (i, 0)),
            pl.BlockSpec((rows, 128), lambda i: (i, 0)),
            pl.BlockSpec((rows, 128), lambda i: (i, 0)),
        ],
        out_specs=[
            pl.BlockSpec((rows, vocab), lambda i: (i, 0)),
            pl.BlockSpec((rows, 128), lambda i: (i, 0)),
        ],
        out_shape=[
            jax.ShapeDtypeStruct((batch, vocab), jnp.float32),
            jax.ShapeDtypeStruct((batch, 128), jnp.int32),
        ],
    )(logits, cval2, ccol2, tcol, mcol)
    return idx[:, :1], probs


def kernel(logits, top_k):
    batch, vocab = logits.shape
    assert batch % _NSUB == 0 and vocab % _GRP == 0 and vocab >= 2 * _GRP
    assert (vocab // _GRP - 1) % 8 == 0
    kmax = min(_KCAP, vocab)
    kidx = jnp.clip(
        jnp.minimum(jnp.asarray(top_k, jnp.int32), vocab) - 1, 0, kmax - 1
    )
    idx_next, probs = _run(logits, kidx)
    return idx_next, probs


# speedup vs baseline: 1.5109x; 1.5109x over previous
import functools

import jax
import jax.numpy as jnp
import numpy as np
from jax import lax
from jax.experimental import pallas as pl
from jax.experimental.pallas import tpu as pltpu
from jax.experimental.pallas import tpu_sc as plsc

_TEMPERATURE = 0.8
_ROT_A = (13, 15, 26, 6)
_ROT_B = (17, 29, 16, 24)
_KEY_HI = np.uint32(0)
_KEY_LO = np.uint32(42)
_TINY = np.float32(1.1754943508222875e-38)
_NEG = np.float32(-np.inf)

_NC = 2
_NS = 16
_NSUB = _NC * _NS
_C = 384
_GRP = 160
_KCAP = 50


def _rotl(x, r):
    return (x << np.uint32(r)) | (x >> np.uint32(32 - r))


def _threefry_bits(flat_u32):
    ks0 = _KEY_HI
    ks1 = _KEY_LO
    ks2 = np.uint32(ks0 ^ ks1 ^ np.uint32(0x1BD11BDA))
    x0 = jnp.zeros_like(flat_u32) + ks0
    x1 = flat_u32 + ks1

    def four(x0, x1, rots):
        for r in rots:
            x0 = x0 + x1
            x1 = _rotl(x1, r)
            x1 = x1 ^ x0
        return x0, x1

    x0, x1 = four(x0, x1, _ROT_A)
    x0 = x0 + ks1
    x1 = x1 + np.uint32(ks2 + np.uint32(1))
    x0, x1 = four(x0, x1, _ROT_B)
    x0 = x0 + ks2
    x1 = x1 + np.uint32(ks0 + np.uint32(2))
    x0, x1 = four(x0, x1, _ROT_A)
    x0 = x0 + ks0
    x1 = x1 + np.uint32(ks1 + np.uint32(3))
    x0, x1 = four(x0, x1, _ROT_B)
    x0 = x0 + ks1
    x1 = x1 + np.uint32(ks2 + np.uint32(4))
    x0, x1 = four(x0, x1, _ROT_A)
    x0 = x0 + ks2
    x1 = x1 + np.uint32(ks0 + np.uint32(5))
    return x0 ^ x1


def _gumbel_from_flat(flat_i32):
    bits = _threefry_bits(flat_i32.astype(jnp.uint32))
    float_bits = (bits >> np.uint32(9)) | np.uint32(0x3F800000)
    floats = jax.lax.bitcast_convert_type(float_bits, jnp.float32) - np.float32(1.0)
    u = jnp.maximum(_TINY, floats * np.float32(np.float32(1.0) - _TINY) + _TINY)
    return -jnp.log(-jnp.log(u))


def _extract_kth(cval, kidx, neg_vec):
    def body(_, carry):
        cnt, tcur, tfin = carry
        mv = [neg_vec] * 4
        for j in range(_C // 16):
            v = cval[pl.ds(j * 16, 16)]
            mv[j % 4] = jnp.maximum(mv[j % 4], jnp.where(v < tcur, v, neg_vec))
        m = jnp.max(jnp.maximum(jnp.maximum(mv[0], mv[1]), jnp.maximum(mv[2], mv[3])))
        cv = [jnp.zeros((16,), jnp.float32)] * 4
        for j in range(_C // 16):
            v = cval[pl.ds(j * 16, 16)]
            cv[j % 4] = cv[j % 4] + (v == m).astype(jnp.float32)
        cc = jnp.sum((cv[0] + cv[1]) + (cv[2] + cv[3]))
        done = cnt > kidx
        newcnt = cnt + cc
        hit = jnp.logical_and(jnp.logical_not(done), newcnt > kidx)
        tfin = jnp.where(hit, m, tfin)
        cnt = jnp.where(done, cnt, newcnt)
        tcur = jnp.where(done, tcur, m)
        return cnt, tcur, tfin

    _, _, t = lax.fori_loop(
        0, _KCAP, body, (np.float32(0.0), np.float32(np.inf), _NEG)
    )
    return t


def _tighten(cval, ccol, kidx, pos_s, tau_s, lane, neg_vec):
    kth = _extract_kth(cval, kidx, neg_vec)
    pos2 = 0
    for j in range(_C // 16):
        v = cval[pl.ds(j * 16, 16)]
        cols = ccol[pl.ds(j * 16, 16)]
        mk = v >= kth
        cj = jnp.sum(mk.astype(jnp.float32)).astype(jnp.int32)

        @pl.when(cj > 0)
        def _(v=v, cols=cols, mk=mk, pos2=pos2):
            plsc.store_compressed(cval.at[pl.ds(pos2, 16)], v, mask=mk)
            plsc.store_compressed(ccol.at[pl.ds(pos2, 16)], cols, mask=mk)

        pos2 = pos2 + cj
    for j in range(_C // 16):
        lidx = lane + np.int32(j * 16)
        v = cval[pl.ds(j * 16, 16)]
        cval[pl.ds(j * 16, 16)] = jnp.where(lidx < pos2, v, neg_vec)
    pos_s[0] = pos2
    tau_s[0] = kth


def _build_sc_select(batch, vocab):
    rows_per = batch // _NSUB
    n_groups = vocab // _GRP
    mesh = plsc.VectorSubcoreMesh(
        core_axis_name="c", subcore_axis_name="s", num_cores=_NC, num_subcores=_NS
    )

    @functools.partial(
        pl.kernel,
        mesh=mesh,
        out_type=[
            jax.ShapeDtypeStruct((batch * _C,), jnp.float32),
            jax.ShapeDtypeStruct((batch * _C,), jnp.int32),
            jax.ShapeDtypeStruct((batch * 16,), jnp.float32),
        ],

        scratch_types=[
            pltpu.VMEM((vocab,), jnp.float32),
            pltpu.VMEM((_C,), jnp.float32),
            pltpu.VMEM((_C,), jnp.int32),
            pltpu.VMEM((16,), jnp.float32),
            pltpu.VMEM((16,), jnp.float32),
            pltpu.SMEM((8,), jnp.int32),
            pltpu.SMEM((8,), jnp.float32),
        ],
        compiler_params=pltpu.CompilerParams(needs_layout_passes=False),
    )
    def sel(
        logits_hbm,
        kvec_hbm,
        val_hbm,
        col_hbm,
        stats_hbm,
        rowbuf,
        cval,
        ccol,
        kvec_v,
        stage_f,
        pos_s,
        tau_s,
    ):
        wid = lax.axis_index("s") * _NC + lax.axis_index("c")
        pltpu.sync_copy(kvec_hbm, kvec_v)
        kidx = jnp.max(kvec_v[...])
        lane = lax.iota(jnp.int32, 16)
        neg_vec = jnp.full((16,), _NEG, jnp.float32)
        zero_vec = jnp.zeros((16,), jnp.int32)

        def row_body(rr, _carry):
            r = wid * rows_per + rr
            pltpu.sync_copy(logits_hbm.at[pl.ds(r * vocab, vocab)], rowbuf)
            for j in range(_C // 16):
                cval[pl.ds(j * 16, 16)] = neg_vec
                ccol[pl.ds(j * 16, 16)] = zero_vec
            for j in range(_GRP // 16):
                off = j * 16
                cval[pl.ds(off, 16)] = rowbuf[pl.ds(off, 16)]
                ccol[pl.ds(off, 16)] = lane + np.int32(off)
            pos_s[0] = np.int32(_GRP)
            tau_s[0] = _NEG
            _tighten(cval, ccol, kidx, pos_s, tau_s, lane, neg_vec)

            def group(gi, _):
                tau = tau_s[0]
                gmaxes = []
                for half in range(4):
                    hbase = (1 + 4 * gi + half) * _GRP
                    mv = [neg_vec] * 4
                    for j in range(_GRP // 16):
                        v = rowbuf[pl.ds(hbase + j * 16, 16)]
                        mv[j % 4] = jnp.maximum(mv[j % 4], v)
                    gmaxes.append(
                        jnp.max(
                            jnp.maximum(
                                jnp.maximum(mv[0], mv[1]), jnp.maximum(mv[2], mv[3])
                            )
                        )
                    )

                for half in range(4):

                    @pl.when(gmaxes[half] >= tau)
                    def _(half=half):
                        base = (1 + 4 * gi + half) * _GRP
                        vs = []
                        mks = []
                        cs = []
                        for j in range(_GRP // 16):
                            v = rowbuf[pl.ds(base + j * 16, 16)]
                            mk = v >= tau
                            vs.append(v)
                            mks.append(mk)
                            cs.append(
                                jnp.sum(mk.astype(jnp.float32)).astype(jnp.int32)
                            )
                        ctot = cs[0]
                        for j in range(1, _GRP // 16):
                            ctot = ctot + cs[j]
                        p0 = pos_s[0]

                        @pl.when(p0 + ctot > _C - 16)
                        def _():
                            _tighten(cval, ccol, kidx, pos_s, tau_s, lane, neg_vec)

                        p = pos_s[0]

                        @pl.when(p + ctot <= _C - 16)
                        def _():
                            off = p
                            for j in range(_GRP // 16):
                                cols = lane + (base + np.int32(j * 16))
                                plsc.store_compressed(
                                    cval.at[pl.ds(off, 16)], vs[j], mask=mks[j]
                                )
                                plsc.store_compressed(
                                    ccol.at[pl.ds(off, 16)], cols, mask=mks[j]
                                )
                                off = off + cs[j]
                            pos_s[0] = off

                return 0

            lax.fori_loop(0, (n_groups - 1) // 4, group, 0)

            mv = [neg_vec] * 4
            for j in range(_C // 16):
                mv[j % 4] = jnp.maximum(mv[j % 4], cval[pl.ds(j * 16, 16)])
            m_row = jnp.max(
                jnp.maximum(jnp.maximum(mv[0], mv[1]), jnp.maximum(mv[2], mv[3]))
            )
            t_row = _extract_kth(cval, kidx, neg_vec)
            stage_f[...] = jnp.where(
                lane == 0, t_row, jnp.where(lane == 1, m_row, np.float32(0.0))
            )
            pltpu.sync_copy(cval, val_hbm.at[pl.ds(r * _C, _C)])
            pltpu.sync_copy(ccol, col_hbm.at[pl.ds(r * _C, _C)])
            pltpu.sync_copy(stage_f, stats_hbm.at[pl.ds(r * 16, 16)])
            return 0

        lax.fori_loop(0, rows_per, row_body, 0)

    return sel


def _tc_finish_body(
    vocab, logits_ref, cval_ref, ccol_ref, tcol_ref, mcol_ref, probs_ref, idx_ref
):
    temp = np.float32(_TEMPERATURE)
    rows = logits_ref.shape[0]
    t_s = tcol_ref[:, :1] / temp
    m_s = mcol_ref[:, :1] / temp

    sv = cval_ref[...] / temp
    mask = sv >= t_s
    e = jnp.where(mask, jnp.exp(sv - m_s), np.float32(0.0))
    s_row = jnp.sum(e, axis=1, keepdims=True)

    col = ccol_ref[...]
    pid = pl.program_id(0)
    row = jax.lax.broadcasted_iota(jnp.int32, col.shape, 0) + pid * rows
    flat = row * vocab + col
    g = _gumbel_from_flat(flat)
    score = jnp.where(mask, sv + g, _NEG)
    sm = jnp.max(score, axis=1, keepdims=True)
    first = jnp.min(
        jnp.where(score == sm, col, np.int32(2**30)), axis=1, keepdims=True
    )
    idx_ref[...] = jnp.broadcast_to(first, idx_ref.shape).astype(jnp.int32)

    scaled = logits_ref[...] / temp
    probs_ref[...] = jnp.where(
        scaled >= t_s, jnp.exp(scaled - m_s) / s_row, np.float32(0.0)
    )


@jax.jit
def _run(logits, kidx):
    batch, vocab = logits.shape
    rows = 8
    sel = _build_sc_select(batch, vocab)
    kvec = jnp.full((16,), kidx, jnp.int32).astype(jnp.float32)
    val, colb, stats = sel(logits.reshape(-1), kvec)
    stats2 = stats.reshape(batch, 16)
    tcol = jnp.broadcast_to(stats2[:, 0:1], (batch, 128))
    mcol = jnp.broadcast_to(stats2[:, 1:2], (batch, 128))
    cval2 = val.reshape(batch, _C)
    ccol2 = colb.reshape(batch, _C)

    probs, idx = pl.pallas_call(
        functools.partial(_tc_finish_body, vocab),
        grid=(batch // rows,),
        in_specs=[
            pl.BlockSpec((rows, vocab), lambda i: (i, 0)),
            pl.BlockSpec((rows, _C), lambda i: (i, 0)),
            pl.BlockSpec((rows, _C), lambda i: (i, 0)),
            pl.BlockSpec((rows, 128), lambda i: (i, 0)),
            pl.BlockSpec((rows, 128), lambda i: (i, 0)),
        ],
        out_specs=[
            pl.BlockSpec((rows, vocab), lambda i: (i, 0)),
            pl.BlockSpec((rows, 128), lambda i: (i, 0)),
        ],
        out_shape=[
            jax.ShapeDtypeStruct((batch, vocab), jnp.float32),
            jax.ShapeDtypeStruct((batch, 128), jnp.int32),
        ],
    )(logits, cval2, ccol2, tcol, mcol)
    return idx[:, :1], probs


def kernel(logits, top_k):
    batch, vocab = logits.shape
    assert batch % _NSUB == 0 and vocab % _GRP == 0 and vocab >= 2 * _GRP
    assert (vocab // _GRP - 1) % 4 == 0
    kmax = min(_KCAP, vocab)
    kidx = jnp.clip(
        jnp.minimum(jnp.asarray(top_k, jnp.int32), vocab) - 1, 0, kmax - 1
    )
    idx_next, probs = _run(logits, kidx)
    return idx_next, probs
